# Initial kernel scaffold; baseline (speedup 1.0000x reference)
import jax
import jax.numpy as jnp
from jax.experimental import pallas as pl


def kernel(x, tgt_x, tgt_edge_index, tgt_edge_type, embed_table, W_et1, W_cross1, b1, W_et3, W_cross3, b3, W_z, b_z, W_g, b_g):
    raise NotImplementedError("write your pallas kernel here")


# speedup vs baseline: 4.3994x; 4.3994x over previous
import functools

import jax
import jax.numpy as jnp
from jax import lax
from jax.experimental import pallas as pl
from jax.experimental.pallas import tpu as pltpu
from jax.experimental.pallas import tpu_sc as plsc

HIDDEN = 256
EMBED = 128
VOCAB = 1000
N = 10000
E = 160000
T = 4
NSRT = 7

NC = 2
NS = 16
NW = NC * NS
CHUNK = 128
NP = 10240
EP = 163840
DUMP = N
ROWS_PER_TILE = NP // NS
CHUNKS_PER_TILE = EP // NW // CHUNK
NBLK = 256
PREC = jax.lax.Precision.HIGHEST


def _mesh():
    return plsc.VectorSubcoreMesh(
        core_axis_name="c", subcore_axis_name="s", num_cores=NC, num_subcores=NS
    )


def _embed_call(tokT, tab):
    @functools.partial(
        pl.kernel,
        out_type=jax.ShapeDtypeStruct((NP, EMBED), jnp.float32),
        mesh=_mesh(),
        scratch_types=[
            pltpu.VMEM((3, CHUNK), jnp.int32),
            pltpu.VMEM((CHUNK, EMBED), jnp.float32),
            pltpu.VMEM((CHUNK, EMBED), jnp.float32),
            pltpu.VMEM((CHUNK, EMBED), jnp.float32),
            pltpu.SemaphoreType.DMA,
        ],
    )
    def k(tok_hbm, tab_hbm, out_hbm, idx_v, g0, g1, g2, sem):
        c = lax.axis_index("c")
        s = lax.axis_index("s")
        wid = c * NS + s

        @pl.loop(wid, NP // CHUNK, step=NW)
        def _(j):
            base = j * CHUNK
            pltpu.sync_copy(tok_hbm.at[pl.ds(base, CHUNK)], idx_v.at[0])
            pltpu.sync_copy(tok_hbm.at[pl.ds(NP + base, CHUNK)], idx_v.at[1])
            pltpu.sync_copy(tok_hbm.at[pl.ds(2 * NP + base, CHUNK)], idx_v.at[2])
            pltpu.async_copy(tab_hbm.at[idx_v.at[0]], g0, sem).wait()
            pltpu.async_copy(tab_hbm.at[idx_v.at[1]], g1, sem).wait()
            pltpu.async_copy(tab_hbm.at[idx_v.at[2]], g2, sem).wait()

            @pl.loop(0, CHUNK)
            def _(r):
                for l in range(0, EMBED, 16):
                    sl = pl.ds(l, 16)
                    g0[r, sl] = g0[r, sl] + g1[r, sl] + g2[r, sl]

            pltpu.sync_copy(g0, out_hbm.at[pl.ds(base, CHUNK)])

    return k(tokT, tab)


def _deg_call(dst_p):
    @functools.partial(
        pl.kernel,
        out_type=jax.ShapeDtypeStruct((NC, NP, 16), jnp.float32),
        mesh=_mesh(),
        scratch_types=[
            pltpu.VMEM_SHARED((NP, 16), jnp.float32),
            pltpu.VMEM((1, CHUNK), jnp.int32),
            pltpu.VMEM((CHUNK, 16), jnp.float32),
            pltpu.SemaphoreType.DMA,
        ],
        compiler_params=pltpu.CompilerParams(use_tc_tiling_on_sc=False),
    )
    def k(dst_hbm, out_hbm, shared_d, idx_v, buf, sem):
        c = lax.axis_index("c")
        s = lax.axis_index("s")

        @pl.loop(0, CHUNK)
        def _(r):
            buf[r] = jnp.zeros((16,), jnp.float32)

        @pl.loop(0, ROWS_PER_TILE // CHUNK)
        def _(kk):
            pltpu.sync_copy(buf, shared_d.at[pl.ds(s * ROWS_PER_TILE + kk * CHUNK, CHUNK)])

        @pl.loop(0, CHUNK)
        def _(r):
            buf[r] = jnp.ones((16,), jnp.float32)

        plsc.subcore_barrier()
        base_t = (c * NS + s) * (CHUNKS_PER_TILE * CHUNK)

        @pl.loop(0, CHUNKS_PER_TILE)
        def _(j):
            pltpu.sync_copy(dst_hbm.at[pl.ds(base_t + j * CHUNK, CHUNK)], idx_v.at[0])
            pltpu.sync_copy(buf, shared_d.at[idx_v.at[0]], add=True)

        plsc.subcore_barrier()
        pltpu.sync_copy(
            shared_d.at[pl.ds(s * ROWS_PER_TILE, ROWS_PER_TILE)],
            out_hbm.at[c, pl.ds(s * ROWS_PER_TILE, ROWS_PER_TILE)],
        )

    return k(dst_p)


def _scatter_call(ht_flat, src_p, et_p, dst_p):
    @functools.partial(
        pl.kernel,
        out_type=jax.ShapeDtypeStruct((NC, NP, EMBED), jnp.float32),
        mesh=_mesh(),
        scratch_types=[
            pltpu.VMEM_SHARED((NP, EMBED), jnp.float32),
            pltpu.VMEM((3, CHUNK), jnp.int32),
            pltpu.VMEM((CHUNK, EMBED), jnp.float32),
            pltpu.SemaphoreType.DMA,
        ],
    )
    def k(ht_hbm, src_hbm, et_hbm, dst_hbm, agg_hbm, shared, idx_v, rows_v, sem):
        c = lax.axis_index("c")
        s = lax.axis_index("s")

        @pl.loop(0, CHUNK)
        def _(r):
            for l in range(0, EMBED, 16):
                rows_v[r, pl.ds(l, 16)] = jnp.zeros((16,), jnp.float32)

        @pl.loop(0, ROWS_PER_TILE // CHUNK)
        def _(kk):
            pltpu.sync_copy(rows_v, shared.at[pl.ds(s * ROWS_PER_TILE + kk * CHUNK, CHUNK)])

        plsc.subcore_barrier()
        base_t = (c * NS + s) * (CHUNKS_PER_TILE * CHUNK)

        @pl.loop(0, CHUNKS_PER_TILE)
        def _(j):
            base = base_t + j * CHUNK
            pltpu.sync_copy(src_hbm.at[pl.ds(base, CHUNK)], idx_v.at[0])
            pltpu.sync_copy(et_hbm.at[pl.ds(base, CHUNK)], idx_v.at[2])
            pltpu.sync_copy(dst_hbm.at[pl.ds(base, CHUNK)], idx_v.at[1])
            for l in range(0, CHUNK, 16):
                sl = pl.ds(l, 16)
                idx_v[0, sl] = idx_v[2, sl] * NP + idx_v[0, sl]
            pltpu.async_copy(ht_hbm.at[idx_v.at[0]], rows_v, sem).wait()
            pltpu.sync_copy(rows_v, shared.at[idx_v.at[1]], add=True)

        plsc.subcore_barrier()
        pltpu.sync_copy(
            shared.at[pl.ds(s * ROWS_PER_TILE, ROWS_PER_TILE)],
            agg_hbm.at[c, pl.ds(s * ROWS_PER_TILE, ROWS_PER_TILE)],
        )

    return k(ht_flat, src_p, et_p, dst_p)


def _edge_gather_call(A, B, src_p, dst_p):
    @functools.partial(
        pl.kernel,
        out_type=jax.ShapeDtypeStruct((EP, 16), jnp.float32),
        mesh=_mesh(),
        scratch_types=[
            pltpu.VMEM((2, CHUNK), jnp.int32),
            pltpu.VMEM((CHUNK, 16), jnp.float32),
            pltpu.VMEM((CHUNK, 16), jnp.float32),
            pltpu.SemaphoreType.DMA,
        ],
        compiler_params=pltpu.CompilerParams(use_tc_tiling_on_sc=False),
    )
    def k(a_hbm, b_hbm, src_hbm, dst_hbm, out_hbm, idx_v, ga, gb, sem):
        c = lax.axis_index("c")
        s = lax.axis_index("s")
        base_t = (c * NS + s) * (CHUNKS_PER_TILE * CHUNK)

        @pl.loop(0, CHUNKS_PER_TILE)
        def _(j):
            base = base_t + j * CHUNK
            pltpu.sync_copy(src_hbm.at[pl.ds(base, CHUNK)], idx_v.at[0])
            pltpu.sync_copy(dst_hbm.at[pl.ds(base, CHUNK)], idx_v.at[1])
            pltpu.async_copy(a_hbm.at[idx_v.at[0]], ga, sem).wait()
            pltpu.async_copy(b_hbm.at[idx_v.at[1]], gb, sem).wait()

            @pl.loop(0, CHUNK)
            def _(r):
                ga[r] = ga[r] + gb[r]

            pltpu.sync_copy(ga, out_hbm.at[pl.ds(base, CHUNK)])

    return k(A, B, src_p, dst_p)


def _ctx_call(x, W1, W3):
    nsteps = 10
    blk = N // nsteps

    def body(x_ref, w1_ref, w3_ref, o1_ref, o3_ref, acc_ref):
        i = pl.program_id(0)

        @pl.when(i == 0)
        def _():
            acc_ref[...] = jnp.zeros_like(acc_ref)

        xb = x_ref[...]
        acc_ref[...] += jnp.sum(xb.reshape(blk // 8, 8, HIDDEN), axis=0)

        @pl.when(i == nsteps - 1)
        def _():
            m = jnp.sum(acc_ref[...], axis=0, keepdims=True) * (1.0 / N)
            o1_ref[...] = jnp.dot(m, w1_ref[...], preferred_element_type=jnp.float32,
                                  precision=PREC)
            o3_ref[...] = jnp.dot(m, w3_ref[...], preferred_element_type=jnp.float32,
                                  precision=PREC)

    return pl.pallas_call(
        body,
        grid=(nsteps,),
        in_specs=[
            pl.BlockSpec((blk, HIDDEN), lambda i: (i, 0)),
            pl.BlockSpec((HIDDEN, EMBED), lambda i: (0, 0)),
            pl.BlockSpec((HIDDEN, EMBED), lambda i: (0, 0)),
        ],
        out_specs=[
            pl.BlockSpec((1, EMBED), lambda i: (0, 0)),
            pl.BlockSpec((1, EMBED), lambda i: (0, 0)),
        ],
        out_shape=[
            jax.ShapeDtypeStruct((1, EMBED), jnp.float32),
            jax.ShapeDtypeStruct((1, EMBED), jnp.float32),
        ],
        scratch_shapes=[pltpu.VMEM((8, HIDDEN), jnp.float32)],
        compiler_params=pltpu.CompilerParams(
            dimension_semantics=("arbitrary",)),
    )(x, W1, W3)


def _mm_call(h, W_et):
    def body(h_ref, w_ref, o_ref):
        hb = h_ref[...]
        for t in range(T):
            o_ref[t] = jnp.dot(hb, w_ref[t], preferred_element_type=jnp.float32,
                               precision=PREC)

    return pl.pallas_call(
        body,
        grid=(NP // NBLK,),
        in_specs=[
            pl.BlockSpec((NBLK, EMBED), lambda i: (i, 0)),
            pl.BlockSpec((T, EMBED, EMBED), lambda i: (0, 0, 0)),
        ],
        out_specs=pl.BlockSpec((T, NBLK, EMBED), lambda i: (0, i, 0)),
        out_shape=jax.ShapeDtypeStruct((T, NP, EMBED), jnp.float32),
    )(h, W_et)


def _combine(agg_ref, deg_ref, ctx_ref, b_ref):
    a = agg_ref[0] + agg_ref[1]
    deg = deg_ref[0, :, 0] + deg_ref[1, :, 0]
    inv = 1.0 / jnp.maximum(deg, 1.0)
    return jnp.maximum(a * inv[:, None] + ctx_ref[...] + b_ref[...], 0.0)


def _combine_mm_call(agg2, deg2, ctx, b, W_et):
    def body(agg_ref, deg_ref, ctx_ref, b_ref, w_ref, o_ref):
        hb = _combine(agg_ref, deg_ref, ctx_ref, b_ref)
        for t in range(T):
            o_ref[t] = jnp.dot(hb, w_ref[t], preferred_element_type=jnp.float32,
                               precision=PREC)

    return pl.pallas_call(
        body,
        grid=(NP // NBLK,),
        in_specs=[
            pl.BlockSpec((NC, NBLK, EMBED), lambda i: (0, i, 0)),
            pl.BlockSpec((NC, NBLK, 16), lambda i: (0, i, 0)),
            pl.BlockSpec((1, EMBED), lambda i: (0, 0)),
            pl.BlockSpec((1, EMBED), lambda i: (0, 0)),
            pl.BlockSpec((T, EMBED, EMBED), lambda i: (0, 0, 0)),
        ],
        out_specs=pl.BlockSpec((T, NBLK, EMBED), lambda i: (0, i, 0)),
        out_shape=jax.ShapeDtypeStruct((T, NP, EMBED), jnp.float32),
    )(agg2, deg2, ctx, b, W_et)


def _final_h_call(agg2, deg2, ctx, b, Wg_a, Wg_b):
    def body(agg_ref, deg_ref, ctx_ref, b_ref, wa_ref, wb_ref,
             h_ref, a_ref, b2_ref):
        hb = _combine(agg_ref, deg_ref, ctx_ref, b_ref)
        h_ref[...] = hb
        a_ref[...] = jnp.dot(hb, wa_ref[...], preferred_element_type=jnp.float32,
                             precision=PREC)
        b2_ref[...] = jnp.dot(hb, wb_ref[...], preferred_element_type=jnp.float32,
                              precision=PREC)

    return pl.pallas_call(
        body,
        grid=(NP // NBLK,),
        in_specs=[
            pl.BlockSpec((NC, NBLK, EMBED), lambda i: (0, i, 0)),
            pl.BlockSpec((NC, NBLK, 16), lambda i: (0, i, 0)),
            pl.BlockSpec((1, EMBED), lambda i: (0, 0)),
            pl.BlockSpec((1, EMBED), lambda i: (0, 0)),
            pl.BlockSpec((EMBED, 16), lambda i: (0, 0)),
            pl.BlockSpec((EMBED, 16), lambda i: (0, 0)),
        ],
        out_specs=[
            pl.BlockSpec((NBLK, EMBED), lambda i: (i, 0)),
            pl.BlockSpec((NBLK, 16), lambda i: (i, 0)),
            pl.BlockSpec((NBLK, 16), lambda i: (i, 0)),
        ],
        out_shape=[
            jax.ShapeDtypeStruct((NP, EMBED), jnp.float32),
            jax.ShapeDtypeStruct((NP, 16), jnp.float32),
            jax.ShapeDtypeStruct((NP, 16), jnp.float32),
        ],
    )(agg2, deg2, ctx, b, Wg_a, Wg_b)


def _node_pred_call(h, W_z, b_z):
    def body(h_ref, wz_ref, bz_ref, o_ref):
        logits = jnp.dot(h_ref[...], wz_ref[...], preferred_element_type=jnp.float32,
                         precision=PREC) + bz_ref[...]
        m = jnp.max(logits, axis=1, keepdims=True)
        sh = logits - m
        lse = jnp.log(jnp.sum(jnp.exp(sh), axis=1, keepdims=True))
        o_ref[...] = sh - lse

    return pl.pallas_call(
        body,
        grid=(NP // NBLK,),
        in_specs=[
            pl.BlockSpec((NBLK, EMBED), lambda i: (i, 0)),
            pl.BlockSpec((EMBED, VOCAB), lambda i: (0, 0)),
            pl.BlockSpec((1, VOCAB), lambda i: (0, 0)),
        ],
        out_specs=pl.BlockSpec((NBLK, VOCAB), lambda i: (i, 0)),
        out_shape=jax.ShapeDtypeStruct((NP, VOCAB), jnp.float32),
    )(h, W_z, b_z)


def _edge_pred_call(eh, bg16):
    blk = 2048

    def body(eh_ref, bg_ref, o_ref):
        v = eh_ref[...] + bg_ref[...]
        col = lax.broadcasted_iota(jnp.int32, (blk, 16), 1)
        valid = col < NSRT
        m = jnp.max(jnp.where(valid, v, -1e30), axis=1, keepdims=True)
        sh = v - m
        ex = jnp.where(valid, jnp.exp(sh), 0.0)
        lse = jnp.log(jnp.sum(ex, axis=1, keepdims=True))
        res = sh - lse
        o_ref[...] = lax.slice(res, (0, 0), (blk, NSRT))

    return pl.pallas_call(
        body,
        grid=(EP // blk,),
        in_specs=[
            pl.BlockSpec((blk, 16), lambda i: (i, 0)),
            pl.BlockSpec((1, 16), lambda i: (0, 0)),
        ],
        out_specs=pl.BlockSpec((blk, NSRT), lambda i: (i, 0)),
        out_shape=jax.ShapeDtypeStruct((EP, NSRT), jnp.float32),
    )(eh, bg16)


def kernel(x, tgt_x, tgt_edge_index, tgt_edge_type, embed_table,
           W_et1, W_cross1, b1, W_et3, W_cross3, b3, W_z, b_z, W_g, b_g):
    tokT = jnp.zeros((3, NP), jnp.int32).at[:, :N].set(
        tgt_x.astype(jnp.int32).T).reshape(3 * NP)
    src = tgt_edge_index[0].astype(jnp.int32)
    dst = tgt_edge_index[1].astype(jnp.int32)
    et = tgt_edge_type.astype(jnp.int32)
    pad = EP - E
    src_p = jnp.concatenate([src, jnp.zeros((pad,), jnp.int32)])
    dst_p = jnp.concatenate([dst, jnp.full((pad,), DUMP, jnp.int32)])
    et_p = jnp.concatenate([et, jnp.zeros((pad,), jnp.int32)])
    b1r = b1.reshape(1, EMBED)
    b3r = b3.reshape(1, EMBED)
    bzr = b_z.reshape(1, VOCAB)
    Wg_a = jnp.pad(W_g[:EMBED], ((0, 0), (0, 16 - NSRT)))
    Wg_b = jnp.pad(W_g[EMBED:], ((0, 0), (0, 16 - NSRT)))
    bg16 = jnp.pad(b_g, (0, 16 - NSRT)).reshape(1, 16)

    h0 = _embed_call(tokT, embed_table)
    deg2 = _deg_call(dst_p)
    ctx1, ctx3 = _ctx_call(x, W_cross1, W_cross3)

    ht = _mm_call(h0, W_et1)
    agg = _scatter_call(ht.reshape(T * NP, EMBED), src_p, et_p, dst_p)
    ht = _combine_mm_call(agg, deg2, ctx1, b1r, W_et1)
    agg = _scatter_call(ht.reshape(T * NP, EMBED), src_p, et_p, dst_p)
    ht = _combine_mm_call(agg, deg2, ctx1, b1r, W_et3)
    agg = _scatter_call(ht.reshape(T * NP, EMBED), src_p, et_p, dst_p)

    h3, A, B = _final_h_call(agg, deg2, ctx3, b3r, Wg_a, Wg_b)
    node_full = _node_pred_call(h3, W_z, bzr)
    eh = _edge_gather_call(A, B, src_p, dst_p)
    ep_full = _edge_pred_call(eh, bg16)

    return node_full[:N], ep_full[:E]


# speedup vs baseline: 5.9296x; 1.3478x over previous
import functools

import jax
import jax.numpy as jnp
from jax import lax
from jax.experimental import pallas as pl
from jax.experimental.pallas import tpu as pltpu
from jax.experimental.pallas import tpu_sc as plsc

HIDDEN = 256
EMBED = 128
VOCAB = 1000
N = 10000
E = 160000
T = 4
NSRT = 7

NC = 2
NS = 16
NW = NC * NS
CHUNK = 128
NP = 10240
EP = 163840
DUMP = N
ROWS_PER_TILE = NP // NS
CHUNKS_PER_TILE = EP // NW // CHUNK
NBLK = 256
PREC = jax.lax.Precision.HIGHEST


def _mesh():
    return plsc.VectorSubcoreMesh(
        core_axis_name="c", subcore_axis_name="s", num_cores=NC, num_subcores=NS
    )


def _embed_call(tok2, tab):
    @functools.partial(
        pl.kernel,
        out_type=jax.ShapeDtypeStruct((NP, EMBED), jnp.float32),
        mesh=_mesh(),
        scratch_types=[
            pltpu.VMEM((8, CHUNK), jnp.int32),
            pltpu.VMEM((CHUNK, EMBED), jnp.float32),
            pltpu.VMEM((CHUNK, EMBED), jnp.float32),
            pltpu.VMEM((CHUNK, EMBED), jnp.float32),
            pltpu.SemaphoreType.DMA,
            pltpu.SemaphoreType.DMA,
            pltpu.SemaphoreType.DMA,
        ],
    )
    def k(tok_hbm, tab_hbm, out_hbm, idx_v, g0, g1, g2, s0, s1, s2):
        c = lax.axis_index("c")
        s = lax.axis_index("s")
        wid = c * NS + s

        @pl.loop(wid, NP // CHUNK, step=NW)
        def _(j):
            base = j * CHUNK
            pltpu.sync_copy(tok_hbm.at[pl.ds(8 * j, 8)], idx_v)
            pltpu.async_copy(tab_hbm.at[idx_v.at[0]], g0, s0)
            pltpu.async_copy(tab_hbm.at[idx_v.at[1]], g1, s1)
            pltpu.async_copy(tab_hbm.at[idx_v.at[2]], g2, s2)
            pltpu.make_async_copy(tab_hbm.at[idx_v.at[0]], g0, s0).wait()
            pltpu.make_async_copy(tab_hbm.at[idx_v.at[1]], g1, s1).wait()
            pltpu.make_async_copy(tab_hbm.at[idx_v.at[2]], g2, s2).wait()

            @pl.loop(0, CHUNK)
            def _(r):
                for l in range(0, EMBED, 16):
                    sl = pl.ds(l, 16)
                    g0[r, sl] = g0[r, sl] + g1[r, sl] + g2[r, sl]

            pltpu.sync_copy(g0, out_hbm.at[pl.ds(base, CHUNK)])

    return k(tok2, tab)


def _deg_call(dst2):
    @functools.partial(
        pl.kernel,
        out_type=jax.ShapeDtypeStruct((NC, NP, 16), jnp.float32),
        mesh=_mesh(),
        scratch_types=[
            pltpu.VMEM_SHARED((NP, 16), jnp.float32),
            pltpu.VMEM((CHUNKS_PER_TILE, CHUNK), jnp.int32),
            pltpu.VMEM((CHUNK, 16), jnp.float32),
            pltpu.SemaphoreType.DMA,
        ],
        compiler_params=pltpu.CompilerParams(use_tc_tiling_on_sc=False),
    )
    def k(dst_hbm, out_hbm, shared_d, di_v, buf, sem):
        c = lax.axis_index("c")
        s = lax.axis_index("s")
        chunk0 = (c * NS + s) * CHUNKS_PER_TILE
        pltpu.async_copy(dst_hbm.at[pl.ds(chunk0, CHUNKS_PER_TILE)], di_v, sem)

        @pl.loop(0, CHUNK)
        def _(r):
            buf[r] = jnp.zeros((16,), jnp.float32)

        @pl.loop(0, ROWS_PER_TILE // CHUNK)
        def _(kk):
            pltpu.sync_copy(buf, shared_d.at[pl.ds(s * ROWS_PER_TILE + kk * CHUNK, CHUNK)])

        @pl.loop(0, CHUNK)
        def _(r):
            buf[r] = jnp.ones((16,), jnp.float32)

        pltpu.make_async_copy(dst_hbm.at[pl.ds(chunk0, CHUNKS_PER_TILE)], di_v, sem).wait()
        plsc.subcore_barrier()

        @pl.loop(0, CHUNKS_PER_TILE)
        def _(j):
            pltpu.sync_copy(buf, shared_d.at[di_v.at[j]], add=True)

        plsc.subcore_barrier()
        pltpu.sync_copy(
            shared_d.at[pl.ds(s * ROWS_PER_TILE, ROWS_PER_TILE)],
            out_hbm.at[c, pl.ds(s * ROWS_PER_TILE, ROWS_PER_TILE)],
        )

    return k(dst2)


def _scatter_call(ht_flat, src2, et2, dst2):
    @functools.partial(
        pl.kernel,
        out_type=jax.ShapeDtypeStruct((NC, NP, EMBED), jnp.float32),
        mesh=_mesh(),
        scratch_types=[
            pltpu.VMEM_SHARED((NP, EMBED), jnp.float32),
            pltpu.VMEM((CHUNKS_PER_TILE, CHUNK), jnp.int32),
            pltpu.VMEM((CHUNKS_PER_TILE, CHUNK), jnp.int32),
            pltpu.VMEM((CHUNKS_PER_TILE, CHUNK), jnp.int32),
            pltpu.VMEM((CHUNK, EMBED), jnp.float32),
            pltpu.VMEM((CHUNK, EMBED), jnp.float32),
            pltpu.SemaphoreType.DMA,
            pltpu.SemaphoreType.DMA,
        ],
    )
    def k(ht_hbm, src_hbm, et_hbm, dst_hbm, agg_hbm,
          shared, gi_v, et_v, di_v, r0, r1, sem0, sem1):
        c = lax.axis_index("c")
        s = lax.axis_index("s")
        chunk0 = (c * NS + s) * CHUNKS_PER_TILE

        pltpu.async_copy(src_hbm.at[pl.ds(chunk0, CHUNKS_PER_TILE)], gi_v, sem1)
        pltpu.async_copy(et_hbm.at[pl.ds(chunk0, CHUNKS_PER_TILE)], et_v, sem1)
        pltpu.async_copy(dst_hbm.at[pl.ds(chunk0, CHUNKS_PER_TILE)], di_v, sem1)

        @pl.loop(0, CHUNK)
        def _(r):
            for l in range(0, EMBED, 16):
                r0[r, pl.ds(l, 16)] = jnp.zeros((16,), jnp.float32)

        @pl.loop(0, ROWS_PER_TILE // CHUNK)
        def _(kk):
            pltpu.sync_copy(r0, shared.at[pl.ds(s * ROWS_PER_TILE + kk * CHUNK, CHUNK)])

        pltpu.make_async_copy(src_hbm.at[pl.ds(chunk0, CHUNKS_PER_TILE)], gi_v, sem1).wait()
        pltpu.make_async_copy(et_hbm.at[pl.ds(chunk0, CHUNKS_PER_TILE)], et_v, sem1).wait()
        pltpu.make_async_copy(dst_hbm.at[pl.ds(chunk0, CHUNKS_PER_TILE)], di_v, sem1).wait()

        @pl.loop(0, CHUNKS_PER_TILE)
        def _(j):
            for l in range(0, CHUNK, 16):
                sl = pl.ds(l, 16)
                gi_v[j, sl] = et_v[j, sl] * NP + gi_v[j, sl]

        plsc.subcore_barrier()

        pltpu.async_copy(ht_hbm.at[gi_v.at[0]], r0, sem0)

        @pl.loop(0, CHUNKS_PER_TILE, step=2)
        def _(j):
            pltpu.make_async_copy(ht_hbm.at[gi_v.at[j]], r0, sem0).wait()
            pltpu.async_copy(ht_hbm.at[gi_v.at[j + 1]], r1, sem1)
            pltpu.sync_copy(r0, shared.at[di_v.at[j]], add=True)
            pltpu.make_async_copy(ht_hbm.at[gi_v.at[j + 1]], r1, sem1).wait()

            @pl.when(j + 2 < CHUNKS_PER_TILE)
            def _():
                pltpu.async_copy(ht_hbm.at[gi_v.at[j + 2]], r0, sem0)

            pltpu.sync_copy(r1, shared.at[di_v.at[j + 1]], add=True)

        plsc.subcore_barrier()
        pltpu.sync_copy(
            shared.at[pl.ds(s * ROWS_PER_TILE, ROWS_PER_TILE)],
            agg_hbm.at[c, pl.ds(s * ROWS_PER_TILE, ROWS_PER_TILE)],
        )

    return k(ht_flat, src2, et2, dst2)


def _edge_gather_call(A, B, src2, dst2):
    @functools.partial(
        pl.kernel,
        out_type=jax.ShapeDtypeStruct((EP // 8, EMBED), jnp.float32),
        mesh=_mesh(),
        scratch_types=[
            pltpu.VMEM((CHUNKS_PER_TILE, CHUNK), jnp.int32),
            pltpu.VMEM((CHUNKS_PER_TILE, CHUNK), jnp.int32),
            pltpu.VMEM((CHUNK, 16), jnp.float32),
            pltpu.VMEM((CHUNK, 16), jnp.float32),
            pltpu.VMEM((CHUNK // 8, EMBED), jnp.float32),
            pltpu.SemaphoreType.DMA,
            pltpu.SemaphoreType.DMA,
            pltpu.SemaphoreType.DMA,
        ],
        compiler_params=pltpu.CompilerParams(use_tc_tiling_on_sc=False),
    )
    def k(a_hbm, b_hbm, src_hbm, dst_hbm, out_hbm,
          si_v, di_v, ga, gb, gc, sema, semb, semo):
        c = lax.axis_index("c")
        s = lax.axis_index("s")
        chunk0 = (c * NS + s) * CHUNKS_PER_TILE

        pltpu.async_copy(src_hbm.at[pl.ds(chunk0, CHUNKS_PER_TILE)], si_v, sema)
        pltpu.async_copy(dst_hbm.at[pl.ds(chunk0, CHUNKS_PER_TILE)], di_v, semb)
        pltpu.make_async_copy(src_hbm.at[pl.ds(chunk0, CHUNKS_PER_TILE)], si_v, sema).wait()
        pltpu.make_async_copy(dst_hbm.at[pl.ds(chunk0, CHUNKS_PER_TILE)], di_v, semb).wait()

        @pl.loop(0, CHUNKS_PER_TILE)
        def _(j):
            pltpu.async_copy(a_hbm.at[si_v.at[j]], ga, sema)
            pltpu.async_copy(b_hbm.at[di_v.at[j]], gb, semb)
            pltpu.make_async_copy(a_hbm.at[si_v.at[j]], ga, sema).wait()
            pltpu.make_async_copy(b_hbm.at[di_v.at[j]], gb, semb).wait()

            @pl.loop(0, CHUNK // 8)
            def _(q):
                for m in range(8):
                    gc[q, pl.ds(m * 16, 16)] = ga[q * 8 + m] + gb[q * 8 + m]

            pltpu.sync_copy(gc, out_hbm.at[pl.ds((chunk0 + j) * (CHUNK // 8), CHUNK // 8)])

    return k(A, B, src2, dst2)


def _ctx_call(x, W1, W3):
    nsteps = 10
    blk = N // nsteps

    def body(x_ref, w1_ref, w3_ref, o1_ref, o3_ref, acc_ref):
        i = pl.program_id(0)

        @pl.when(i == 0)
        def _():
            acc_ref[...] = jnp.zeros_like(acc_ref)

        xb = x_ref[...]
        acc_ref[...] += jnp.sum(xb.reshape(blk // 8, 8, HIDDEN), axis=0)

        @pl.when(i == nsteps - 1)
        def _():
            m = jnp.sum(acc_ref[...], axis=0, keepdims=True) * (1.0 / N)
            o1_ref[...] = jnp.dot(m, w1_ref[...], preferred_element_type=jnp.float32,
                                  precision=PREC)
            o3_ref[...] = jnp.dot(m, w3_ref[...], preferred_element_type=jnp.float32,
                                  precision=PREC)

    return pl.pallas_call(
        body,
        grid=(nsteps,),
        in_specs=[
            pl.BlockSpec((blk, HIDDEN), lambda i: (i, 0)),
            pl.BlockSpec((HIDDEN, EMBED), lambda i: (0, 0)),
            pl.BlockSpec((HIDDEN, EMBED), lambda i: (0, 0)),
        ],
        out_specs=[
            pl.BlockSpec((1, EMBED), lambda i: (0, 0)),
            pl.BlockSpec((1, EMBED), lambda i: (0, 0)),
        ],
        out_shape=[
            jax.ShapeDtypeStruct((1, EMBED), jnp.float32),
            jax.ShapeDtypeStruct((1, EMBED), jnp.float32),
        ],
        scratch_shapes=[pltpu.VMEM((8, HIDDEN), jnp.float32)],
        compiler_params=pltpu.CompilerParams(
            dimension_semantics=("arbitrary",)),
    )(x, W1, W3)


def _mm_call(h, W_et):
    def body(h_ref, w_ref, o_ref):
        hb = h_ref[...]
        for t in range(T):
            o_ref[t] = jnp.dot(hb, w_ref[t], preferred_element_type=jnp.float32,
                               precision=PREC)

    return pl.pallas_call(
        body,
        grid=(NP // NBLK,),
        in_specs=[
            pl.BlockSpec((NBLK, EMBED), lambda i: (i, 0)),
            pl.BlockSpec((T, EMBED, EMBED), lambda i: (0, 0, 0)),
        ],
        out_specs=pl.BlockSpec((T, NBLK, EMBED), lambda i: (0, i, 0)),
        out_shape=jax.ShapeDtypeStruct((T, NP, EMBED), jnp.float32),
    )(h, W_et)


def _combine(agg_ref, deg_ref, ctx_ref, b_ref):
    a = agg_ref[0] + agg_ref[1]
    deg = deg_ref[0, :, 0] + deg_ref[1, :, 0]
    inv = 1.0 / jnp.maximum(deg, 1.0)
    return jnp.maximum(a * inv[:, None] + ctx_ref[...] + b_ref[...], 0.0)


def _combine_mm_call(agg2, deg2, ctx, b, W_et):
    def body(agg_ref, deg_ref, ctx_ref, b_ref, w_ref, o_ref):
        hb = _combine(agg_ref, deg_ref, ctx_ref, b_ref)
        for t in range(T):
            o_ref[t] = jnp.dot(hb, w_ref[t], preferred_element_type=jnp.float32,
                               precision=PREC)

    return pl.pallas_call(
        body,
        grid=(NP // NBLK,),
        in_specs=[
            pl.BlockSpec((NC, NBLK, EMBED), lambda i: (0, i, 0)),
            pl.BlockSpec((NC, NBLK, 16), lambda i: (0, i, 0)),
            pl.BlockSpec((1, EMBED), lambda i: (0, 0)),
            pl.BlockSpec((1, EMBED), lambda i: (0, 0)),
            pl.BlockSpec((T, EMBED, EMBED), lambda i: (0, 0, 0)),
        ],
        out_specs=pl.BlockSpec((T, NBLK, EMBED), lambda i: (0, i, 0)),
        out_shape=jax.ShapeDtypeStruct((T, NP, EMBED), jnp.float32),
    )(agg2, deg2, ctx, b, W_et)


def _final_h_call(agg2, deg2, ctx, b, Wg_a, Wg_b):
    def body(agg_ref, deg_ref, ctx_ref, b_ref, wa_ref, wb_ref,
             h_ref, a_ref, b2_ref):
        hb = _combine(agg_ref, deg_ref, ctx_ref, b_ref)
        h_ref[...] = hb
        a_ref[...] = jnp.dot(hb, wa_ref[...], preferred_element_type=jnp.float32,
                             precision=PREC)
        b2_ref[...] = jnp.dot(hb, wb_ref[...], preferred_element_type=jnp.float32,
                              precision=PREC)

    return pl.pallas_call(
        body,
        grid=(NP // NBLK,),
        in_specs=[
            pl.BlockSpec((NC, NBLK, EMBED), lambda i: (0, i, 0)),
            pl.BlockSpec((NC, NBLK, 16), lambda i: (0, i, 0)),
            pl.BlockSpec((1, EMBED), lambda i: (0, 0)),
            pl.BlockSpec((1, EMBED), lambda i: (0, 0)),
            pl.BlockSpec((EMBED, 16), lambda i: (0, 0)),
            pl.BlockSpec((EMBED, 16), lambda i: (0, 0)),
        ],
        out_specs=[
            pl.BlockSpec((NBLK, EMBED), lambda i: (i, 0)),
            pl.BlockSpec((NBLK, 16), lambda i: (i, 0)),
            pl.BlockSpec((NBLK, 16), lambda i: (i, 0)),
        ],
        out_shape=[
            jax.ShapeDtypeStruct((NP, EMBED), jnp.float32),
            jax.ShapeDtypeStruct((NP, 16), jnp.float32),
            jax.ShapeDtypeStruct((NP, 16), jnp.float32),
        ],
    )(agg2, deg2, ctx, b, Wg_a, Wg_b)


def _node_pred_call(h, W_z, b_z):
    def body(h_ref, wz_ref, bz_ref, o_ref):
        logits = jnp.dot(h_ref[...], wz_ref[...], preferred_element_type=jnp.float32,
                         precision=PREC) + bz_ref[...]
        m = jnp.max(logits, axis=1, keepdims=True)
        sh = logits - m
        lse = jnp.log(jnp.sum(jnp.exp(sh), axis=1, keepdims=True))
        o_ref[...] = sh - lse

    return pl.pallas_call(
        body,
        grid=(NP // NBLK,),
        in_specs=[
            pl.BlockSpec((NBLK, EMBED), lambda i: (i, 0)),
            pl.BlockSpec((EMBED, VOCAB), lambda i: (0, 0)),
            pl.BlockSpec((1, VOCAB), lambda i: (0, 0)),
        ],
        out_specs=pl.BlockSpec((NBLK, VOCAB), lambda i: (i, 0)),
        out_shape=jax.ShapeDtypeStruct((NP, VOCAB), jnp.float32),
    )(h, W_z, b_z)


def _edge_pred_call(eh2, bg128):
    blk = 512

    def body(eh_ref, bg_ref, o_ref):
        v = eh_ref[...] + bg_ref[...]
        col16 = lax.broadcasted_iota(jnp.int32, (blk, EMBED), 1) % 16
        valid = col16 < NSRT
        m = jnp.max(jnp.where(valid, v, -1e30), axis=1, keepdims=True)
        ex = jnp.where(valid, jnp.exp(v - m), 0.0)
        gi = lax.broadcasted_iota(jnp.int32, (EMBED, EMBED), 0) // 16
        gj = lax.broadcasted_iota(jnp.int32, (EMBED, EMBED), 1) // 16
        G = (gi == gj).astype(jnp.float32)
        gsum = jnp.dot(ex, G, preferred_element_type=jnp.float32, precision=PREC)
        o_ref[...] = (v - m) - jnp.log(gsum)

    return pl.pallas_call(
        body,
        grid=(EP // 8 // blk,),
        in_specs=[
            pl.BlockSpec((blk, EMBED), lambda i: (i, 0)),
            pl.BlockSpec((1, EMBED), lambda i: (0, 0)),
        ],
        out_specs=pl.BlockSpec((blk, EMBED), lambda i: (i, 0)),
        out_shape=jax.ShapeDtypeStruct((EP // 8, EMBED), jnp.float32),
    )(eh2, bg128)


def kernel(x, tgt_x, tgt_edge_index, tgt_edge_type, embed_table,
           W_et1, W_cross1, b1, W_et3, W_cross3, b3, W_z, b_z, W_g, b_g):
    tok2 = (jnp.zeros((8, NP), jnp.int32).at[:3, :N].set(tgt_x.astype(jnp.int32).T)
            .reshape(8, NP // CHUNK, CHUNK).transpose(1, 0, 2)
            .reshape(8 * NP // CHUNK, CHUNK))
    src = tgt_edge_index[0].astype(jnp.int32)
    dst = tgt_edge_index[1].astype(jnp.int32)
    et = tgt_edge_type.astype(jnp.int32)
    pad = EP - E
    src2 = jnp.concatenate([src, jnp.zeros((pad,), jnp.int32)]).reshape(EP // CHUNK, CHUNK)
    dst2 = jnp.concatenate([dst, jnp.full((pad,), DUMP, jnp.int32)]).reshape(EP // CHUNK, CHUNK)
    et2 = jnp.concatenate([et, jnp.zeros((pad,), jnp.int32)]).reshape(EP // CHUNK, CHUNK)
    b1r = b1.reshape(1, EMBED)
    b3r = b3.reshape(1, EMBED)
    bzr = b_z.reshape(1, VOCAB)
    Wg_a = jnp.pad(W_g[:EMBED], ((0, 0), (0, 16 - NSRT)))
    Wg_b = jnp.pad(W_g[EMBED:], ((0, 0), (0, 16 - NSRT)))
    bg128 = jnp.tile(jnp.pad(b_g, (0, 16 - NSRT)), 8).reshape(1, EMBED)

    h0 = _embed_call(tok2, embed_table)
    deg2 = _deg_call(dst2)
    ctx1, ctx3 = _ctx_call(x, W_cross1, W_cross3)

    ht = _mm_call(h0, W_et1)
    agg = _scatter_call(ht.reshape(T * NP, EMBED), src2, et2, dst2)
    ht = _combine_mm_call(agg, deg2, ctx1, b1r, W_et1)
    agg = _scatter_call(ht.reshape(T * NP, EMBED), src2, et2, dst2)
    ht = _combine_mm_call(agg, deg2, ctx1, b1r, W_et3)
    agg = _scatter_call(ht.reshape(T * NP, EMBED), src2, et2, dst2)

    h3, A, B = _final_h_call(agg, deg2, ctx3, b3r, Wg_a, Wg_b)
    node_full = _node_pred_call(h3, W_z, bzr)
    eh2 = _edge_gather_call(A, B, src2, dst2)
    ep_pack = _edge_pred_call(eh2, bg128)

    edge_pred = ep_pack.reshape(EP // 8, 8, 16)[:, :, :NSRT].reshape(EP, NSRT)[:E]
    return node_full[:N], edge_pred
